# baseline (speedup 1.0000x reference)
import jax
import jax.numpy as jnp
import numpy as np
from jax import lax
from jax.experimental import pallas as pl
from jax.experimental.pallas import tpu as pltpu
from jax.experimental.pallas import tpu_sc as plsc

B = 4
E = 80000
M = B * E
C = 128

BH = 2
MH = BH * E

NW = 32
PER_W = MH // NW
KCH = 40
NCHUNK = PER_W // KCH
NGRP = 1
GCH = NCHUNK // NGRP

BLK = 3200
NEB = E // BLK


def _sc_gather_body(table, idxw, o1, o2, o3, o4, ivall,
                    b00, b01, b02, b03, b10, b11, b12, b13,
                    sg0, sg1, sw0, sw1):
    wid = lax.axis_index("s") * 2 + lax.axis_index("c")
    base0 = wid * PER_W

    bufs = ((b00, b01, b02, b03), (b10, b11, b12, b13))
    sgs = (sg0, sg1)
    sws = (sw0, sw1)
    ohs = (o1, o2, o3, o4)

    def group(g, carry):
        pltpu.sync_copy(idxw.at[wid, g], ivall)

        def fire_gathers(lci, s):
            return [pltpu.async_copy(table.at[ivall.at[lci * 4 + j]],
                                     bufs[s][j], sgs[s]) for j in range(4)]

        def fire_writes(lci, s):
            base = base0 + (g * GCH + lci) * KCH
            for j in range(4):
                pltpu.async_copy(bufs[s][j], ohs[j].at[pl.ds(base, KCH)],
                                 sws[s])

        def wait_writes(s):
            for j in range(4):
                pltpu.make_async_copy(bufs[s][j], ohs[j].at[pl.ds(0, KCH)],
                                      sws[s]).wait()

        def run_chunk(lci, s):
            cps = fire_gathers(lci, s)
            for cp in cps:
                cp.wait()
            fire_writes(lci, s)

        run_chunk(0, 0)
        run_chunk(1, 1)

        def pair(k, c2):
            for s in (0, 1):
                wait_writes(s)
                run_chunk(2 + 2 * k + s, s)
            return c2

        lax.fori_loop(0, (GCH - 2) // 2, pair, 0)
        if (GCH - 2) % 2 == 1:
            wait_writes(0)
            run_chunk(GCH - 1, 0)
        wait_writes(0)
        wait_writes(1)
        return carry

    lax.fori_loop(0, NGRP, group, 0)


def _make_sc_gather(dtype):
    mesh = plsc.VectorSubcoreMesh(core_axis_name="c", subcore_axis_name="s")
    out_t = [jax.ShapeDtypeStruct((MH, C), dtype)] * 4
    scratch = ([pltpu.VMEM((GCH * 4, KCH), jnp.int32)]
               + [pltpu.VMEM((KCH, C), dtype)] * 8
               + [pltpu.SemaphoreType.DMA] * 4)
    return pl.kernel(_sc_gather_body, mesh=mesh, out_type=out_t,
                     scratch_types=scratch)


_HI = np.uint32(0xFFFF0000)


def _pack2(a, b):
    ah = lax.bitcast_convert_type(a.astype(jnp.bfloat16).astype(jnp.float32),
                                  jnp.uint32)
    bh = lax.bitcast_convert_type(b.astype(jnp.bfloat16).astype(jnp.float32),
                                  jnp.uint32)
    return lax.bitcast_convert_type((ah & _HI) | (bh >> 16), jnp.int32)


def _unpack_hi(p):
    u = lax.bitcast_convert_type(p, jnp.uint32)
    return lax.bitcast_convert_type(u & _HI, jnp.float32)


def _unpack_lo(p):
    u = lax.bitcast_convert_type(p, jnp.uint32)
    return lax.bitcast_convert_type(u << 16, jnp.float32)


def _combo(a1, a2, a3, a4):
    return [a1 + a3, a2 + a4, jnp.abs(a1 - a3), jnp.abs(a2 - a4)]


def _bf(xs):
    return [x.astype(jnp.bfloat16) for x in xs]


def _pack_body(up, dn, t1, fuo, fdo):
    u = up[...].reshape(C, BLK).T
    d = dn[...].reshape(C, BLK).T
    t1[...] = _pack2(u, d)
    fuo[...] = u
    fdo[...] = d


def _conv1_body(fu, fd, p1, p2, p3, p4, w1, wz, b1, bz, t2):
    hi = [_unpack_hi(p[...]) for p in (p1, p2, p3, p4)]
    lo = [_unpack_lo(p[...]) for p in (p1, p2, p3, p4)]
    gy = jnp.concatenate([fu[...].astype(jnp.bfloat16)] + _bf(_combo(*hi)),
                         axis=1)
    gz = jnp.concatenate([fd[...].astype(jnp.bfloat16)] + _bf(_combo(*lo)),
                         axis=1)
    y1 = jnp.dot(gy, w1[...], preferred_element_type=jnp.float32) + b1[...]
    z = jnp.dot(gz, wz[...], preferred_element_type=jnp.float32) + bz[...]
    t2[...] = _pack2(y1, z)


def _stats_epilogue(i, y, acc1, acc2, scale, shift):
    @pl.when(i == 0)
    def _():
        acc1[...] = jnp.zeros_like(acc1)
        acc2[...] = jnp.zeros_like(acc2)

    acc1[...] += jnp.sum(y, axis=0, keepdims=True)
    acc2[...] += jnp.sum(y * y, axis=0, keepdims=True)

    @pl.when(i == NEB - 1)
    def _():
        mean = acc1[...] * (1.0 / E)
        var = acc2[...] * (1.0 / E) - mean * mean
        rstd = lax.rsqrt(var + 1e-5)
        scale[...] = rstd.reshape(1, 1, C)
        shift[...] = (-mean * rstd).reshape(1, 1, C)


def _conv2_body(t2r, q1, q2, q3, q4, w, y2, scale, shift, acc1, acc2):
    i = pl.program_id(1)
    y1f0 = _unpack_hi(t2r[...])
    z = _unpack_lo(t2r[...])
    qs = [_unpack_hi(q[...]) for q in (q1, q2, q3, q4)]
    g = jnp.concatenate([y1f0.astype(jnp.bfloat16)] + _bf(_combo(*qs)),
                        axis=1)
    y = jnp.dot(g, w[...], preferred_element_type=jnp.float32) + z
    y2[...] = y
    _stats_epilogue(i, y, acc1, acc2, scale, shift)


def _x1(y2val, s2, h2):
    return jnp.maximum(y2val * s2.reshape(1, C) + h2.reshape(1, C), 0.0)


def _conv3_body(y2r, a1, a2, a3, a4, s2, h2, w, bias,
                y3, scale, shift, acc1, acc2):
    i = pl.program_id(1)
    x0 = _x1(y2r[...], s2[...], h2[...])
    xs = [_x1(a[...], s2[...], h2[...]) for a in (a1, a2, a3, a4)]
    g = jnp.concatenate([x0.astype(jnp.bfloat16)] + _bf(_combo(*xs)), axis=1)
    y = jnp.dot(g, w[...], preferred_element_type=jnp.float32) + bias[...]
    y3[...] = y.astype(jnp.bfloat16)
    _stats_epilogue(i, y, acc1, acc2, scale, shift)


def _final_body(y3r, y2r, s2, h2, scale, shift, out):
    r = jnp.maximum(y3r[...] * scale[...].reshape(1, C)
                    + shift[...].reshape(1, C)
                    + _x1(y2r[...], s2[...], h2[...]), 0.0)
    out[...] = r.T.reshape(1, C, BLK)


def _row1():
    return pl.BlockSpec((BLK, C), lambda i: (i, 0))


def _row2():
    return pl.BlockSpec((BLK, C), lambda b, i: (b * NEB + i, 0))


def _w1_spec(k):
    return pl.BlockSpec((k, C), lambda i: (0, 0))


def _w2_spec(k):
    return pl.BlockSpec((k, C), lambda b, i: (0, 0))


def _stat_spec():
    return pl.BlockSpec((1, 1, C), lambda b, i: (b, 0, 0))


_STAT_SHAPE = jax.ShapeDtypeStruct((BH, 1, C), jnp.float32)
_ROW_F32 = jax.ShapeDtypeStruct((MH, C), jnp.float32)
_ROW_BF16 = jax.ShapeDtypeStruct((MH, C), jnp.bfloat16)


def _pack_call(h, from_up, from_down):
    spec = pl.BlockSpec((1, C, BLK),
                        lambda i, hh=h: (hh * BH + i // NEB, 0, i % NEB))
    return pl.pallas_call(
        _pack_body, grid=(MH // BLK,),
        in_specs=[spec, spec],
        out_specs=[_row1(), _row1(), _row1()],
        out_shape=[jax.ShapeDtypeStruct((MH, C), jnp.int32),
                   _ROW_F32, _ROW_F32],
    )(from_up, from_down)


def _conv1_call(fu, fd, p, w1, wz, b1, bz):
    return pl.pallas_call(
        _conv1_body, grid=(MH // BLK,),
        in_specs=[_row1()] * 6
                 + [_w1_spec(5 * C), _w1_spec(5 * C),
                    _w1_spec(1), _w1_spec(1)],
        out_specs=_row1(),
        out_shape=jax.ShapeDtypeStruct((MH, C), jnp.int32),
    )(fu, fd, *p, w1, wz, b1, bz)


def _conv2_call(t2, q, wc):
    return pl.pallas_call(
        _conv2_body, grid=(BH, NEB),
        in_specs=[_row2()] * 5 + [_w2_spec(5 * C)],
        out_specs=[_row2(), _stat_spec(), _stat_spec()],
        out_shape=[_ROW_F32, _STAT_SHAPE, _STAT_SHAPE],
        scratch_shapes=[pltpu.VMEM((1, C), jnp.float32),
                        pltpu.VMEM((1, C), jnp.float32)],
    )(t2, *q, wc)


def _conv3_call(y2, a, s2, h2, wc, bias):
    return pl.pallas_call(
        _conv3_body, grid=(BH, NEB),
        in_specs=([_row2()] * 5 + [_stat_spec(), _stat_spec()]
                  + [_w2_spec(5 * C), _w2_spec(1)]),
        out_specs=[_row2(), _stat_spec(), _stat_spec()],
        out_shape=[_ROW_BF16, _STAT_SHAPE, _STAT_SHAPE],
        scratch_shapes=[pltpu.VMEM((1, C), jnp.float32),
                        pltpu.VMEM((1, C), jnp.float32)],
    )(y2, *a, s2, h2, wc, bias)


def _final_call(y3, y2, s2, h2, scale, shift):
    return pl.pallas_call(
        _final_body, grid=(BH, NEB),
        in_specs=[_row2(), _row2(), _stat_spec(), _stat_spec(),
                  _stat_spec(), _stat_spec()],
        out_specs=pl.BlockSpec((1, C, BLK), lambda b, i: (b, 0, i)),
        out_shape=jax.ShapeDtypeStruct((BH, C, E), jnp.float32),
    )(y3, y2, s2, h2, scale, shift)


def kernel(from_up, from_down, gemm_edges, W_up, b_up, W1, b1, W2, b2):
    def wcat(W, cols):
        return jnp.concatenate([W[:, cs, k].T for (cs, k) in cols],
                               axis=0).astype(jnp.bfloat16)

    full = slice(0, C)
    lo, hi = slice(0, C), slice(C, 2 * C)
    taps5 = [0, 1, 2, 3, 4]
    wc1 = wcat(W_up, [(full, k) for k in taps5])
    wcz = wcat(W1, [(hi, k) for k in taps5])
    wc2 = wcat(W1, [(lo, k) for k in taps5])
    wc3 = wcat(W2, [(full, k) for k in taps5])
    bu = b_up.reshape(1, C)
    bz = b1.reshape(1, C)
    b2r = b2.reshape(1, C)

    sc_i = _make_sc_gather(jnp.int32)
    sc_f = _make_sc_gather(jnp.float32)

    H = B // BH
    idxw = []
    for h in range(H):
        sl = slice(h * BH, (h + 1) * BH)
        ge = (gemm_edges[sl].astype(jnp.int32)
              + (jnp.arange(BH, dtype=jnp.int32) * E)[:, None, None])
        idxw.append(ge.reshape(MH, 4).T
                    .reshape(4, NW, NGRP, GCH, KCH)
                    .transpose(1, 2, 3, 0, 4)
                    .reshape(NW, NGRP, GCH * 4, KCH))

    packed = [_pack_call(h, from_up, from_down) for h in range(H)]
    p = [sc_i(packed[h][0], idxw[h]) for h in range(H)]
    t2 = [_conv1_call(packed[h][1], packed[h][2], p[h], wc1, wcz, bu, bz)
          for h in range(H)]
    q = [sc_i(t2[h], idxw[h]) for h in range(H)]
    y2s = [_conv2_call(t2[h], q[h], wc2) for h in range(H)]
    a = [sc_f(y2s[h][0], idxw[h]) for h in range(H)]
    y3s = [_conv3_call(y2s[h][0], a[h], y2s[h][1], y2s[h][2], wc3, b2r)
           for h in range(H)]
    out = [_final_call(y3s[h][0], y2s[h][0], y2s[h][1], y2s[h][2],
                       y3s[h][1], y3s[h][2]) for h in range(H)]
    return jnp.concatenate(out, axis=0)
